# baseline (speedup 1.0000x reference)
import functools

import jax
import jax.numpy as jnp
from jax import lax
from jax.experimental import pallas as pl
from jax.experimental.pallas import tpu as pltpu
from jax.experimental.pallas import tpu_sc as plsc

N = 10000
E = 320000
NC = 2
NS = 16
NW = NC * NS
CHUNK = 80
NCHUNKS = E // CHUNK
CPT = NCHUNKS // NW
PAIRS = CPT // 2
ROWS_PER_TILE = 624
TAIL_ROWS = N - NS * ROWS_PER_TILE

MB = 1000

_MESH = plsc.VectorSubcoreMesh(core_axis_name="c", subcore_axis_name="s",
                               num_cores=NC, num_subcores=NS)


def _row_copy(mk_src, mk_dst, sid):
    rb = sid * ROWS_PER_TILE
    pltpu.sync_copy(mk_src(rb, ROWS_PER_TILE), mk_dst(rb, ROWS_PER_TILE))

    @pl.when(sid == 0)
    def _():
        base = NS * ROWS_PER_TILE
        pltpu.sync_copy(mk_src(base, TAIL_ROWS), mk_dst(base, TAIL_ROWS))


@functools.partial(
    pl.kernel,
    mesh=_MESH,
    out_type=jax.ShapeDtypeStruct((NC, N, 16), jnp.float32),
    scratch_types=[
        pltpu.VMEM((CPT, CHUNK), jnp.int32),
        pltpu.VMEM((CHUNK, 16), jnp.float32),
        pltpu.VMEM_SHARED((N, 16), jnp.float32),
    ],
    compiler_params=pltpu.CompilerParams(use_tc_tiling_on_sc=False),
)
def _deg_kernel(dst_hbm, ones_hbm, zeros_hbm, out_hbm, didx2, ones_v, acc):
    cid = lax.axis_index("c")
    sid = lax.axis_index("s")
    w = sid * NC + cid
    pltpu.sync_copy(ones_hbm, ones_v)
    pltpu.sync_copy(dst_hbm.at[pl.ds(CPT * w, CPT)], didx2)
    _row_copy(lambda o, s: zeros_hbm.at[pl.ds(o, s)],
              lambda o, s: acc.at[pl.ds(o, s)], sid)
    plsc.subcore_barrier()

    @pl.loop(0, CPT)
    def _(i):
        pltpu.sync_copy(ones_v, acc.at[didx2.at[i]], add=True)

    plsc.subcore_barrier()
    _row_copy(lambda o, s: acc.at[pl.ds(o, s)],
              lambda o, s: out_hbm.at[cid, pl.ds(o, s)], sid)


def _make_agg(C):
    @functools.partial(
        pl.kernel,
        mesh=_MESH,
        out_type=jax.ShapeDtypeStruct((NC, N, C), jnp.float32),
        scratch_types=[
            pltpu.VMEM((CPT, CHUNK), jnp.int32),
            pltpu.VMEM((CPT, CHUNK), jnp.int32),
            pltpu.VMEM((CHUNK, C), jnp.float32),
            pltpu.VMEM((CHUNK, C), jnp.float32),
            pltpu.VMEM_SHARED((N, C), jnp.float32),
            pltpu.SemaphoreType.DMA,
            pltpu.SemaphoreType.DMA,
            pltpu.SemaphoreType.DMA,
            pltpu.SemaphoreType.DMA,
        ],
        compiler_params=pltpu.CompilerParams(use_tc_tiling_on_sc=False),
    )
    def agg_kernel(z_hbm, src_hbm, dst_hbm, zeros_hbm, out_hbm,
                   sidx2, didx2, rows0, rows1, acc, gsem0, gsem1, ssem0, ssem1):
        cid = lax.axis_index("c")
        sid = lax.axis_index("s")
        w = sid * NC + cid
        pltpu.sync_copy(src_hbm.at[pl.ds(CPT * w, CPT)], sidx2)
        pltpu.sync_copy(dst_hbm.at[pl.ds(CPT * w, CPT)], didx2)
        _row_copy(lambda o, s: zeros_hbm.at[pl.ds(o, s)],
                  lambda o, s: acc.at[pl.ds(o, s)], sid)
        plsc.subcore_barrier()

        rows = (rows0, rows1)
        gsems = (gsem0, gsem1)
        ssems = (ssem0, ssem1)

        def gather(i, b):
            pltpu.async_copy(z_hbm.at[sidx2.at[i]], rows[b], gsems[b])

        def wait_g(b):
            pltpu.make_async_copy(z_hbm.at[pl.ds(0, CHUNK)],
                                  rows[b], gsems[b]).wait()

        def scat(i, b):
            pltpu.async_copy(rows[b], acc.at[didx2.at[i]], ssems[b],
                             add=True)

        def wait_s(b):
            pltpu.make_async_copy(rows[b], acc.at[didx2.at[0]],
                                  ssems[b]).wait()

        gather(0, 0)
        gather(1, 1)

        @pl.loop(0, PAIRS)
        def _(j):
            i0 = 2 * j
            wait_g(0)
            scat(i0, 0)
            wait_g(1)
            scat(i0 + 1, 1)
            wait_s(0)
            gather(i0 + 2, 0)

            @pl.when(j < PAIRS - 1)
            def _():
                wait_s(1)
                gather(i0 + 3, 1)

        wait_g(0)
        scat(CPT - 1, 0)
        wait_s(0)
        wait_s(1)

        plsc.subcore_barrier()
        _row_copy(lambda o, s: acc.at[pl.ds(o, s)],
                  lambda o, s: out_hbm.at[cid, pl.ds(o, s)], sid)

    return agg_kernel


_agg128 = _make_agg(128)
_agg64 = _make_agg(64)


def _dinv_block(degp):
    deg = degp[0, :, 0:1] + degp[1, :, 0:1] + 1.0
    return lax.rsqrt(deg)


def _mm_scale_body(x_ref, w_ref, degp_ref, o_ref):
    o_ref[...] = jnp.dot(x_ref[...], w_ref[...],
                         preferred_element_type=jnp.float32) \
        * _dinv_block(degp_ref[...])


def _tc_mm_scale(x, W, degp):
    return pl.pallas_call(
        _mm_scale_body,
        grid=(N // MB,),
        in_specs=[
            pl.BlockSpec((MB, x.shape[1]), lambda i: (i, 0)),
            pl.BlockSpec(W.shape, lambda i: (0, 0)),
            pl.BlockSpec((NC, MB, 16), lambda i: (0, i, 0)),
        ],
        out_specs=pl.BlockSpec((MB, W.shape[1]), lambda i: (i, 0)),
        out_shape=jax.ShapeDtypeStruct((N, W.shape[1]), jnp.float32),
    )(x, W, degp)


def _layer2_body(z1_ref, p_ref, degp_ref, b1_ref, w2_ref, o_ref):
    dinv = _dinv_block(degp_ref[...])
    agg = z1_ref[...] + p_ref[0] + p_ref[1]
    h = jnp.maximum(agg * dinv + b1_ref[...], 0.0)
    o_ref[...] = jnp.dot(h, w2_ref[...],
                         preferred_element_type=jnp.float32) * dinv


def _tc_layer2(z1, p, degp, b1, W2):
    return pl.pallas_call(
        _layer2_body,
        grid=(N // MB,),
        in_specs=[
            pl.BlockSpec((MB, 128), lambda i: (i, 0)),
            pl.BlockSpec((NC, MB, 128), lambda i: (0, i, 0)),
            pl.BlockSpec((NC, MB, 16), lambda i: (0, i, 0)),
            pl.BlockSpec((1, 128), lambda i: (0, 0)),
            pl.BlockSpec((128, 64), lambda i: (0, 0)),
        ],
        out_specs=pl.BlockSpec((MB, 64), lambda i: (i, 0)),
        out_shape=jax.ShapeDtypeStruct((N, 64), jnp.float32),
    )(z1, p, degp, b1, W2)


def _final_body(z2_ref, q_ref, degp_ref, b2_ref, o_ref):
    dinv = _dinv_block(degp_ref[...])
    out2 = (z2_ref[...] + q_ref[0] + q_ref[1]) * dinv + b2_ref[...]
    m = jnp.max(out2, axis=1, keepdims=True)
    e = out2 - m
    lse = jnp.log(jnp.sum(jnp.exp(e), axis=1, keepdims=True))
    o_ref[...] = e - lse


def _tc_final(z2, q, degp, b2):
    return pl.pallas_call(
        _final_body,
        grid=(N // MB,),
        in_specs=[
            pl.BlockSpec((MB, 64), lambda i: (i, 0)),
            pl.BlockSpec((NC, MB, 64), lambda i: (0, i, 0)),
            pl.BlockSpec((NC, MB, 16), lambda i: (0, i, 0)),
            pl.BlockSpec((1, 64), lambda i: (0, 0)),
        ],
        out_specs=pl.BlockSpec((MB, 64), lambda i: (i, 0)),
        out_shape=jax.ShapeDtypeStruct((N, 64), jnp.float32),
    )(z2, q, degp, b2)


def kernel(x, edge_index, W1, b1, W2, b2):
    src = edge_index[0].reshape(NCHUNKS, CHUNK)
    dst = edge_index[1].reshape(NCHUNKS, CHUNK)
    ones16 = jnp.ones((CHUNK, 16), jnp.float32)
    zeros16 = jnp.zeros((N, 16), jnp.float32)
    zeros128 = jnp.zeros((N, 128), jnp.float32)
    zeros64 = jnp.zeros((N, 64), jnp.float32)

    degp = _deg_kernel(dst, ones16, zeros16)
    z1 = _tc_mm_scale(x, W1, degp)
    p = _agg128(z1, src, dst, zeros128)
    z2 = _tc_layer2(z1, p, degp, b1.reshape(1, 128), W2)
    q = _agg64(z2, src, dst, zeros64)
    return _tc_final(z2, q, degp, b2.reshape(1, 64))


# speedup vs baseline: 1.1545x; 1.1545x over previous
import functools

import jax
import jax.numpy as jnp
from jax import lax
from jax.experimental import pallas as pl
from jax.experimental.pallas import tpu as pltpu
from jax.experimental.pallas import tpu_sc as plsc

N = 10000
E = 320000
NC = 2
NS = 16
NW = NC * NS
CHUNK = 80
NCHUNKS = E // CHUNK
CPT = NCHUNKS // NW
PAIRS = CPT // 2
ROWS_PER_TILE = 624
TAIL_ROWS = N - NS * ROWS_PER_TILE

MB = 1000

_MESH = plsc.VectorSubcoreMesh(core_axis_name="c", subcore_axis_name="s",
                               num_cores=NC, num_subcores=NS)


def _row_copy(mk_src, mk_dst, sid):
    rb = sid * ROWS_PER_TILE
    pltpu.sync_copy(mk_src(rb, ROWS_PER_TILE), mk_dst(rb, ROWS_PER_TILE))

    @pl.when(sid == 0)
    def _():
        base = NS * ROWS_PER_TILE
        pltpu.sync_copy(mk_src(base, TAIL_ROWS), mk_dst(base, TAIL_ROWS))


@functools.partial(
    pl.kernel,
    mesh=_MESH,
    out_type=jax.ShapeDtypeStruct((NC, N, 16), jnp.float32),
    scratch_types=[
        pltpu.VMEM((CPT, CHUNK), jnp.int32),
        pltpu.VMEM((CHUNK, 16), jnp.float32),
        pltpu.VMEM_SHARED((N, 16), jnp.float32),
    ],
    compiler_params=pltpu.CompilerParams(use_tc_tiling_on_sc=False),
)
def _deg_kernel(dst_hbm, ones_hbm, zeros_hbm, out_hbm, didx2, ones_v, acc):
    cid = lax.axis_index("c")
    sid = lax.axis_index("s")
    w = sid * NC + cid
    pltpu.sync_copy(ones_hbm, ones_v)
    pltpu.sync_copy(dst_hbm.at[pl.ds(CPT * w, CPT)], didx2)
    _row_copy(lambda o, s: zeros_hbm.at[pl.ds(o, s)],
              lambda o, s: acc.at[pl.ds(o, s)], sid)
    plsc.subcore_barrier()

    @pl.loop(0, CPT)
    def _(i):
        pltpu.sync_copy(ones_v, acc.at[didx2.at[i]], add=True)

    plsc.subcore_barrier()
    _row_copy(lambda o, s: acc.at[pl.ds(o, s)],
              lambda o, s: out_hbm.at[cid, pl.ds(o, s)], sid)


def _make_agg(C):
    @functools.partial(
        pl.kernel,
        mesh=_MESH,
        out_type=jax.ShapeDtypeStruct((NC, N, C), jnp.float32),
        scratch_types=[
            pltpu.VMEM((CPT, CHUNK), jnp.int32),
            pltpu.VMEM((CPT, CHUNK), jnp.int32),
            pltpu.VMEM((CHUNK, C), jnp.float32),
            pltpu.VMEM((CHUNK, C), jnp.float32),
            pltpu.VMEM_SHARED((N, C), jnp.float32),
            pltpu.SemaphoreType.DMA,
            pltpu.SemaphoreType.DMA,
        ],
        compiler_params=pltpu.CompilerParams(use_tc_tiling_on_sc=False),
    )
    def agg_kernel(z_hbm, src_hbm, dst_hbm, zeros_hbm, out_hbm,
                   sidx2, didx2, rows0, rows1, acc, gsem0, gsem1):
        cid = lax.axis_index("c")
        sid = lax.axis_index("s")
        w = sid * NC + cid
        pltpu.sync_copy(src_hbm.at[pl.ds(CPT * w, CPT)], sidx2)
        pltpu.sync_copy(dst_hbm.at[pl.ds(CPT * w, CPT)], didx2)
        _row_copy(lambda o, s: zeros_hbm.at[pl.ds(o, s)],
                  lambda o, s: acc.at[pl.ds(o, s)], sid)
        plsc.subcore_barrier()

        rows = (rows0, rows1)
        gsems = (gsem0, gsem1)

        def gather(i, b):
            pltpu.async_copy(z_hbm.at[sidx2.at[i]], rows[b], gsems[b])

        def wait_g(b):
            pltpu.make_async_copy(z_hbm.at[pl.ds(0, CHUNK)],
                                  rows[b], gsems[b]).wait()

        def scat(i, b):
            pltpu.sync_copy(rows[b], acc.at[didx2.at[i]], add=True)

        gather(0, 0)

        @pl.loop(0, PAIRS)
        def _(j):
            i0 = 2 * j
            gather(i0 + 1, 1)
            wait_g(0)
            scat(i0, 0)
            gather(i0 + 2, 0)
            wait_g(1)
            scat(i0 + 1, 1)

        wait_g(0)
        scat(CPT - 1, 0)

        plsc.subcore_barrier()
        _row_copy(lambda o, s: acc.at[pl.ds(o, s)],
                  lambda o, s: out_hbm.at[cid, pl.ds(o, s)], sid)

    return agg_kernel


_agg128 = _make_agg(128)
_agg64 = _make_agg(64)


def _dinv_block(degp):
    deg = degp[0, :, 0:1] + degp[1, :, 0:1] + 1.0
    return lax.rsqrt(deg)


def _mm_scale_body(x_ref, w_ref, degp_ref, o_ref):
    o_ref[...] = jnp.dot(x_ref[...], w_ref[...],
                         preferred_element_type=jnp.float32) \
        * _dinv_block(degp_ref[...])


def _tc_mm_scale(x, W, degp):
    return pl.pallas_call(
        _mm_scale_body,
        grid=(N // MB,),
        in_specs=[
            pl.BlockSpec((MB, x.shape[1]), lambda i: (i, 0)),
            pl.BlockSpec(W.shape, lambda i: (0, 0)),
            pl.BlockSpec((NC, MB, 16), lambda i: (0, i, 0)),
        ],
        out_specs=pl.BlockSpec((MB, W.shape[1]), lambda i: (i, 0)),
        out_shape=jax.ShapeDtypeStruct((N, W.shape[1]), jnp.float32),
    )(x, W, degp)


def _layer2_body(z1_ref, p_ref, degp_ref, b1_ref, w2_ref, o_ref):
    dinv = _dinv_block(degp_ref[...])
    agg = z1_ref[...] + p_ref[0] + p_ref[1]
    h = jnp.maximum(agg * dinv + b1_ref[...], 0.0)
    o_ref[...] = jnp.dot(h, w2_ref[...],
                         preferred_element_type=jnp.float32) * dinv


def _tc_layer2(z1, p, degp, b1, W2):
    return pl.pallas_call(
        _layer2_body,
        grid=(N // MB,),
        in_specs=[
            pl.BlockSpec((MB, 128), lambda i: (i, 0)),
            pl.BlockSpec((NC, MB, 128), lambda i: (0, i, 0)),
            pl.BlockSpec((NC, MB, 16), lambda i: (0, i, 0)),
            pl.BlockSpec((1, 128), lambda i: (0, 0)),
            pl.BlockSpec((128, 64), lambda i: (0, 0)),
        ],
        out_specs=pl.BlockSpec((MB, 64), lambda i: (i, 0)),
        out_shape=jax.ShapeDtypeStruct((N, 64), jnp.float32),
    )(z1, p, degp, b1, W2)


def _final_body(z2_ref, q_ref, degp_ref, b2_ref, o_ref):
    dinv = _dinv_block(degp_ref[...])
    out2 = (z2_ref[...] + q_ref[0] + q_ref[1]) * dinv + b2_ref[...]
    m = jnp.max(out2, axis=1, keepdims=True)
    e = out2 - m
    lse = jnp.log(jnp.sum(jnp.exp(e), axis=1, keepdims=True))
    o_ref[...] = e - lse


def _tc_final(z2, q, degp, b2):
    return pl.pallas_call(
        _final_body,
        grid=(N // MB,),
        in_specs=[
            pl.BlockSpec((MB, 64), lambda i: (i, 0)),
            pl.BlockSpec((NC, MB, 64), lambda i: (0, i, 0)),
            pl.BlockSpec((NC, MB, 16), lambda i: (0, i, 0)),
            pl.BlockSpec((1, 64), lambda i: (0, 0)),
        ],
        out_specs=pl.BlockSpec((MB, 64), lambda i: (i, 0)),
        out_shape=jax.ShapeDtypeStruct((N, 64), jnp.float32),
    )(z2, q, degp, b2)


def kernel(x, edge_index, W1, b1, W2, b2):
    src = edge_index[0].reshape(NCHUNKS, CHUNK)
    dst = edge_index[1].reshape(NCHUNKS, CHUNK)
    ones16 = jnp.ones((CHUNK, 16), jnp.float32)
    zeros16 = jnp.zeros((N, 16), jnp.float32)
    zeros128 = jnp.zeros((N, 128), jnp.float32)
    zeros64 = jnp.zeros((N, 64), jnp.float32)

    degp = _deg_kernel(dst, ones16, zeros16)
    z1 = _tc_mm_scale(x, W1, degp)
    p = _agg128(z1, src, dst, zeros128)
    z2 = _tc_layer2(z1, p, degp, b1.reshape(1, 128), W2)
    q = _agg64(z2, src, dst, zeros64)
    return _tc_final(z2, q, degp, b2.reshape(1, 64))


# speedup vs baseline: 1.2472x; 1.0802x over previous
import functools

import jax
import jax.numpy as jnp
from jax import lax
from jax.experimental import pallas as pl
from jax.experimental.pallas import tpu as pltpu
from jax.experimental.pallas import tpu_sc as plsc

N = 10000
E = 320000
NC = 2
NS = 16
NW = NC * NS
CHUNK = 80
NCHUNKS = E // CHUNK
CPT = NCHUNKS // NW
PAIRS = CPT // 2
ROWS_PER_TILE = 624
TAIL_ROWS = N - NS * ROWS_PER_TILE

MB = 1000

_MESH = plsc.VectorSubcoreMesh(core_axis_name="c", subcore_axis_name="s",
                               num_cores=NC, num_subcores=NS)


def _row_copy(mk_src, mk_dst, sid):
    rb = sid * ROWS_PER_TILE
    pltpu.sync_copy(mk_src(rb, ROWS_PER_TILE), mk_dst(rb, ROWS_PER_TILE))

    @pl.when(sid == 0)
    def _():
        base = NS * ROWS_PER_TILE
        pltpu.sync_copy(mk_src(base, TAIL_ROWS), mk_dst(base, TAIL_ROWS))


@functools.partial(
    pl.kernel,
    mesh=_MESH,
    out_type=jax.ShapeDtypeStruct((NC, N, 16), jnp.float32),
    scratch_types=[
        pltpu.VMEM((CPT, CHUNK), jnp.int32),
        pltpu.VMEM((CHUNK, 16), jnp.float32),
        pltpu.VMEM_SHARED((N, 16), jnp.float32),
    ],
    compiler_params=pltpu.CompilerParams(use_tc_tiling_on_sc=False),
)
def _deg_kernel(dst_hbm, ones_hbm, zeros_hbm, out_hbm, didx2, ones_v, acc):
    cid = lax.axis_index("c")
    sid = lax.axis_index("s")
    w = sid * NC + cid
    pltpu.sync_copy(ones_hbm, ones_v)
    pltpu.sync_copy(dst_hbm.at[pl.ds(CPT * w, CPT)], didx2)
    _row_copy(lambda o, s: zeros_hbm.at[pl.ds(o, s)],
              lambda o, s: acc.at[pl.ds(o, s)], sid)
    plsc.subcore_barrier()

    @pl.loop(0, CPT)
    def _(i):
        pltpu.sync_copy(ones_v, acc.at[didx2.at[i]], add=True)

    plsc.subcore_barrier()
    _row_copy(lambda o, s: acc.at[pl.ds(o, s)],
              lambda o, s: out_hbm.at[cid, pl.ds(o, s)], sid)


def _make_agg(C):
    @functools.partial(
        pl.kernel,
        mesh=_MESH,
        out_type=jax.ShapeDtypeStruct((NC, N, C), jnp.bfloat16),
        scratch_types=[
            pltpu.VMEM((CPT, CHUNK), jnp.int32),
            pltpu.VMEM((CPT, CHUNK), jnp.int32),
            pltpu.VMEM((CHUNK, C), jnp.bfloat16),
            pltpu.VMEM((CHUNK, C), jnp.bfloat16),
            pltpu.VMEM_SHARED((N, C), jnp.bfloat16),
            pltpu.SemaphoreType.DMA,
            pltpu.SemaphoreType.DMA,
        ],
        compiler_params=pltpu.CompilerParams(use_tc_tiling_on_sc=False),
    )
    def agg_kernel(z_hbm, src_hbm, dst_hbm, zeros_hbm, out_hbm,
                   sidx2, didx2, rows0, rows1, acc, gsem0, gsem1):
        cid = lax.axis_index("c")
        sid = lax.axis_index("s")
        w = sid * NC + cid
        pltpu.sync_copy(src_hbm.at[pl.ds(CPT * w, CPT)], sidx2)
        pltpu.sync_copy(dst_hbm.at[pl.ds(CPT * w, CPT)], didx2)
        _row_copy(lambda o, s: zeros_hbm.at[pl.ds(o, s)],
                  lambda o, s: acc.at[pl.ds(o, s)], sid)
        plsc.subcore_barrier()

        rows = (rows0, rows1)
        gsems = (gsem0, gsem1)

        def gather(i, b):
            pltpu.async_copy(z_hbm.at[sidx2.at[i]], rows[b], gsems[b])

        def wait_g(b):
            pltpu.make_async_copy(z_hbm.at[pl.ds(0, CHUNK)],
                                  rows[b], gsems[b]).wait()

        def scat(i, b):
            pltpu.sync_copy(rows[b], acc.at[didx2.at[i]], add=True)

        gather(0, 0)

        @pl.loop(0, PAIRS)
        def _(j):
            i0 = 2 * j
            gather(i0 + 1, 1)
            wait_g(0)
            scat(i0, 0)
            gather(i0 + 2, 0)
            wait_g(1)
            scat(i0 + 1, 1)

        wait_g(0)
        scat(CPT - 1, 0)

        plsc.subcore_barrier()
        _row_copy(lambda o, s: acc.at[pl.ds(o, s)],
                  lambda o, s: out_hbm.at[cid, pl.ds(o, s)], sid)

    return agg_kernel


_agg128 = _make_agg(128)
_agg64 = _make_agg(64)


def _dinv_block(degp):
    deg = degp[0, :, 0:1] + degp[1, :, 0:1] + 1.0
    return lax.rsqrt(deg)


def _mm_scale_body(x_ref, w_ref, degp_ref, o_ref):
    o_ref[...] = (jnp.dot(x_ref[...], w_ref[...],
                          preferred_element_type=jnp.float32)
                  * _dinv_block(degp_ref[...])).astype(jnp.bfloat16)


def _tc_mm_scale(x, W, degp):
    return pl.pallas_call(
        _mm_scale_body,
        grid=(N // MB,),
        in_specs=[
            pl.BlockSpec((MB, x.shape[1]), lambda i: (i, 0)),
            pl.BlockSpec(W.shape, lambda i: (0, 0)),
            pl.BlockSpec((NC, MB, 16), lambda i: (0, i, 0)),
        ],
        out_specs=pl.BlockSpec((MB, W.shape[1]), lambda i: (i, 0)),
        out_shape=jax.ShapeDtypeStruct((N, W.shape[1]), jnp.bfloat16),
    )(x, W, degp)


def _layer2_body(z1_ref, p_ref, degp_ref, b1_ref, w2_ref, o_ref):
    dinv = _dinv_block(degp_ref[...])
    agg = (z1_ref[...].astype(jnp.float32) + p_ref[0].astype(jnp.float32)
           + p_ref[1].astype(jnp.float32))
    h = jnp.maximum(agg * dinv + b1_ref[...], 0.0)
    o_ref[...] = (jnp.dot(h, w2_ref[...],
                          preferred_element_type=jnp.float32)
                  * dinv).astype(jnp.bfloat16)


def _tc_layer2(z1, p, degp, b1, W2):
    return pl.pallas_call(
        _layer2_body,
        grid=(N // MB,),
        in_specs=[
            pl.BlockSpec((MB, 128), lambda i: (i, 0)),
            pl.BlockSpec((NC, MB, 128), lambda i: (0, i, 0)),
            pl.BlockSpec((NC, MB, 16), lambda i: (0, i, 0)),
            pl.BlockSpec((1, 128), lambda i: (0, 0)),
            pl.BlockSpec((128, 64), lambda i: (0, 0)),
        ],
        out_specs=pl.BlockSpec((MB, 64), lambda i: (i, 0)),
        out_shape=jax.ShapeDtypeStruct((N, 64), jnp.bfloat16),
    )(z1, p, degp, b1, W2)


def _final_body(z2_ref, q_ref, degp_ref, b2_ref, o_ref):
    dinv = _dinv_block(degp_ref[...])
    out2 = (z2_ref[...].astype(jnp.float32) + q_ref[0].astype(jnp.float32)
            + q_ref[1].astype(jnp.float32)) * dinv + b2_ref[...]
    m = jnp.max(out2, axis=1, keepdims=True)
    e = out2 - m
    lse = jnp.log(jnp.sum(jnp.exp(e), axis=1, keepdims=True))
    o_ref[...] = e - lse


def _tc_final(z2, q, degp, b2):
    return pl.pallas_call(
        _final_body,
        grid=(N // MB,),
        in_specs=[
            pl.BlockSpec((MB, 64), lambda i: (i, 0)),
            pl.BlockSpec((NC, MB, 64), lambda i: (0, i, 0)),
            pl.BlockSpec((NC, MB, 16), lambda i: (0, i, 0)),
            pl.BlockSpec((1, 64), lambda i: (0, 0)),
        ],
        out_specs=pl.BlockSpec((MB, 64), lambda i: (i, 0)),
        out_shape=jax.ShapeDtypeStruct((N, 64), jnp.float32),
    )(z2, q, degp, b2)


def kernel(x, edge_index, W1, b1, W2, b2):
    src = edge_index[0].reshape(NCHUNKS, CHUNK)
    dst = edge_index[1].reshape(NCHUNKS, CHUNK)
    ones16 = jnp.ones((CHUNK, 16), jnp.float32)
    zeros16 = jnp.zeros((N, 16), jnp.float32)
    zeros128 = jnp.zeros((N, 128), jnp.bfloat16)
    zeros64 = jnp.zeros((N, 64), jnp.bfloat16)

    degp = _deg_kernel(dst, ones16, zeros16)
    z1 = _tc_mm_scale(x, W1, degp)
    p = _agg128(z1, src, dst, zeros128)
    z2 = _tc_layer2(z1, p, degp, b1.reshape(1, 128), W2)
    q = _agg64(z2, src, dst, zeros64)
    return _tc_final(z2, q, degp, b2.reshape(1, 64))


# speedup vs baseline: 1.2932x; 1.0369x over previous
import functools

import jax
import jax.numpy as jnp
from jax import lax
from jax.experimental import pallas as pl
from jax.experimental.pallas import tpu as pltpu
from jax.experimental.pallas import tpu_sc as plsc

N = 10000
E = 320000
NC = 2
NS = 16
NW = NC * NS
CHUNK = 80
NCHUNKS = E // CHUNK
CPT = NCHUNKS // NW
EPW = E // NW
PAIRS = CPT // 2
ROWS_PER_TILE = 624
TAIL_ROWS = N - NS * ROWS_PER_TILE

MB = 1000

_MESH = plsc.VectorSubcoreMesh(core_axis_name="c", subcore_axis_name="s",
                               num_cores=NC, num_subcores=NS)


def _row_copy(mk_src, mk_dst, sid):
    rb = sid * ROWS_PER_TILE
    pltpu.sync_copy(mk_src(rb, ROWS_PER_TILE), mk_dst(rb, ROWS_PER_TILE))

    @pl.when(sid == 0)
    def _():
        base = NS * ROWS_PER_TILE
        pltpu.sync_copy(mk_src(base, TAIL_ROWS), mk_dst(base, TAIL_ROWS))


@functools.partial(
    pl.kernel,
    mesh=_MESH,
    out_type=jax.ShapeDtypeStruct((NC, N, 16), jnp.float32),
    scratch_types=[
        pltpu.VMEM((EPW,), jnp.int32),
        pltpu.VMEM((CHUNK, 16), jnp.float32),
        pltpu.VMEM_SHARED((N, 16), jnp.float32),
    ],
    compiler_params=pltpu.CompilerParams(use_tc_tiling_on_sc=False),
)
def _deg_kernel(ei_hbm, ones_hbm, zeros_hbm, out_hbm, didx, ones_v, acc):
    cid = lax.axis_index("c")
    sid = lax.axis_index("s")
    w = sid * NC + cid
    pltpu.sync_copy(ones_hbm, ones_v)
    pltpu.sync_copy(ei_hbm.at[1, pl.ds(EPW * w, EPW)], didx)
    _row_copy(lambda o, s: zeros_hbm.at[pl.ds(o, s)],
              lambda o, s: acc.at[pl.ds(o, s)], sid)
    plsc.subcore_barrier()

    @pl.loop(0, CPT)
    def _(i):
        pltpu.sync_copy(ones_v, acc.at[didx.at[pl.ds(i * CHUNK, CHUNK)]],
                        add=True)

    plsc.subcore_barrier()
    _row_copy(lambda o, s: acc.at[pl.ds(o, s)],
              lambda o, s: out_hbm.at[cid, pl.ds(o, s)], sid)


def _make_agg(C):
    @functools.partial(
        pl.kernel,
        mesh=_MESH,
        out_type=jax.ShapeDtypeStruct((NC, N, C), jnp.bfloat16),
        scratch_types=[
            pltpu.VMEM((EPW,), jnp.int32),
            pltpu.VMEM((EPW,), jnp.int32),
            pltpu.VMEM((CHUNK, C), jnp.bfloat16),
            pltpu.VMEM((CHUNK, C), jnp.bfloat16),
            pltpu.VMEM_SHARED((N, C), jnp.bfloat16),
            pltpu.SemaphoreType.DMA,
            pltpu.SemaphoreType.DMA,
        ],
        compiler_params=pltpu.CompilerParams(use_tc_tiling_on_sc=False),
    )
    def agg_kernel(z_hbm, ei_hbm, zeros_hbm, out_hbm,
                   sidx, didx, rows0, rows1, acc, gsem0, gsem1):
        cid = lax.axis_index("c")
        sid = lax.axis_index("s")
        w = sid * NC + cid
        pltpu.sync_copy(ei_hbm.at[0, pl.ds(EPW * w, EPW)], sidx)
        pltpu.sync_copy(ei_hbm.at[1, pl.ds(EPW * w, EPW)], didx)
        _row_copy(lambda o, s: zeros_hbm.at[pl.ds(o, s)],
                  lambda o, s: acc.at[pl.ds(o, s)], sid)
        plsc.subcore_barrier()

        rows = (rows0, rows1)
        gsems = (gsem0, gsem1)

        def gather(i, b):
            pltpu.async_copy(z_hbm.at[sidx.at[pl.ds(i * CHUNK, CHUNK)]],
                             rows[b], gsems[b])

        def wait_g(b):
            pltpu.make_async_copy(z_hbm.at[pl.ds(0, CHUNK)],
                                  rows[b], gsems[b]).wait()

        def scat(i, b):
            pltpu.sync_copy(rows[b], acc.at[didx.at[pl.ds(i * CHUNK, CHUNK)]],
                            add=True)

        gather(0, 0)

        @pl.loop(0, PAIRS)
        def _(j):
            i0 = 2 * j
            gather(i0 + 1, 1)
            wait_g(0)
            scat(i0, 0)
            gather(i0 + 2, 0)
            wait_g(1)
            scat(i0 + 1, 1)

        wait_g(0)
        scat(CPT - 1, 0)

        plsc.subcore_barrier()
        _row_copy(lambda o, s: acc.at[pl.ds(o, s)],
                  lambda o, s: out_hbm.at[cid, pl.ds(o, s)], sid)

    return agg_kernel


_agg128 = _make_agg(128)
_agg64 = _make_agg(64)


def _dinv_block(degp):
    deg = degp[0, :, 0:1] + degp[1, :, 0:1] + 1.0
    return lax.rsqrt(deg)


def _mm_scale_body(x_ref, w_ref, degp_ref, o_ref):
    o_ref[...] = (jnp.dot(x_ref[...], w_ref[...],
                          preferred_element_type=jnp.float32)
                  * _dinv_block(degp_ref[...])).astype(jnp.bfloat16)


def _tc_mm_scale(x, W, degp):
    return pl.pallas_call(
        _mm_scale_body,
        grid=(N // MB,),
        in_specs=[
            pl.BlockSpec((MB, x.shape[1]), lambda i: (i, 0)),
            pl.BlockSpec(W.shape, lambda i: (0, 0)),
            pl.BlockSpec((NC, MB, 16), lambda i: (0, i, 0)),
        ],
        out_specs=pl.BlockSpec((MB, W.shape[1]), lambda i: (i, 0)),
        out_shape=jax.ShapeDtypeStruct((N, W.shape[1]), jnp.bfloat16),
    )(x, W, degp)


def _layer2_body(z1_ref, p_ref, degp_ref, b1_ref, w2_ref, o_ref):
    dinv = _dinv_block(degp_ref[...])
    agg = (z1_ref[...].astype(jnp.float32) + p_ref[0].astype(jnp.float32)
           + p_ref[1].astype(jnp.float32))
    h = jnp.maximum(agg * dinv + b1_ref[...], 0.0)
    o_ref[...] = (jnp.dot(h, w2_ref[...],
                          preferred_element_type=jnp.float32)
                  * dinv).astype(jnp.bfloat16)


def _tc_layer2(z1, p, degp, b1, W2):
    return pl.pallas_call(
        _layer2_body,
        grid=(N // MB,),
        in_specs=[
            pl.BlockSpec((MB, 128), lambda i: (i, 0)),
            pl.BlockSpec((NC, MB, 128), lambda i: (0, i, 0)),
            pl.BlockSpec((NC, MB, 16), lambda i: (0, i, 0)),
            pl.BlockSpec((1, 128), lambda i: (0, 0)),
            pl.BlockSpec((128, 64), lambda i: (0, 0)),
        ],
        out_specs=pl.BlockSpec((MB, 64), lambda i: (i, 0)),
        out_shape=jax.ShapeDtypeStruct((N, 64), jnp.bfloat16),
    )(z1, p, degp, b1, W2)


def _final_body(z2_ref, q_ref, degp_ref, b2_ref, o_ref):
    dinv = _dinv_block(degp_ref[...])
    out2 = (z2_ref[...].astype(jnp.float32) + q_ref[0].astype(jnp.float32)
            + q_ref[1].astype(jnp.float32)) * dinv + b2_ref[...]
    m = jnp.max(out2, axis=1, keepdims=True)
    e = out2 - m
    lse = jnp.log(jnp.sum(jnp.exp(e), axis=1, keepdims=True))
    o_ref[...] = e - lse


def _tc_final(z2, q, degp, b2):
    return pl.pallas_call(
        _final_body,
        grid=(N // MB,),
        in_specs=[
            pl.BlockSpec((MB, 64), lambda i: (i, 0)),
            pl.BlockSpec((NC, MB, 64), lambda i: (0, i, 0)),
            pl.BlockSpec((NC, MB, 16), lambda i: (0, i, 0)),
            pl.BlockSpec((1, 64), lambda i: (0, 0)),
        ],
        out_specs=pl.BlockSpec((MB, 64), lambda i: (i, 0)),
        out_shape=jax.ShapeDtypeStruct((N, 64), jnp.float32),
    )(z2, q, degp, b2)


def kernel(x, edge_index, W1, b1, W2, b2):
    ones16 = jnp.ones((CHUNK, 16), jnp.float32)
    zeros16 = jnp.zeros((N, 16), jnp.float32)
    zeros128 = jnp.zeros((N, 128), jnp.bfloat16)
    zeros64 = jnp.zeros((N, 64), jnp.bfloat16)

    degp = _deg_kernel(edge_index, ones16, zeros16)
    z1 = _tc_mm_scale(x, W1, degp)
    p = _agg128(z1, edge_index, zeros128)
    z2 = _tc_layer2(z1, p, degp, b1.reshape(1, 128), W2)
    q = _agg64(z2, edge_index, zeros64)
    return _tc_final(z2, q, degp, b2.reshape(1, 64))


# speedup vs baseline: 1.5860x; 1.2264x over previous
import functools

import jax
import jax.numpy as jnp
from jax import lax
from jax.experimental import pallas as pl
from jax.experimental.pallas import tpu as pltpu
from jax.experimental.pallas import tpu_sc as plsc

N = 10000
E = 320000
NC = 2
NS = 16
NW = NC * NS
CHUNK = 80
NCHUNKS = E // CHUNK
CPT = NCHUNKS // NW
EPW = E // NW
PAIRS = CPT // 2
ROWS_PER_TILE = 624
TAIL_ROWS = N - NS * ROWS_PER_TILE

MB = 2000

_MESH = plsc.VectorSubcoreMesh(core_axis_name="c", subcore_axis_name="s",
                               num_cores=NC, num_subcores=NS)


def _row_copy(mk_src, mk_dst, sid):
    rb = sid * ROWS_PER_TILE
    pltpu.sync_copy(mk_src(rb, ROWS_PER_TILE), mk_dst(rb, ROWS_PER_TILE))

    @pl.when(sid == 0)
    def _():
        base = NS * ROWS_PER_TILE
        pltpu.sync_copy(mk_src(base, TAIL_ROWS), mk_dst(base, TAIL_ROWS))


@functools.partial(
    pl.kernel,
    mesh=_MESH,
    out_type=jax.ShapeDtypeStruct((NC, N, 16), jnp.float32),
    scratch_types=[
        pltpu.VMEM((EPW,), jnp.int32),
        pltpu.VMEM((CHUNK, 16), jnp.float32),
        pltpu.VMEM_SHARED((N, 16), jnp.float32),
        pltpu.SemaphoreType.DMA,
    ],
    compiler_params=pltpu.CompilerParams(use_tc_tiling_on_sc=False),
)
def _deg_kernel(ei_hbm, ones_hbm, zeros_hbm, out_hbm, didx, ones_v, acc,
                dsem):
    cid = lax.axis_index("c")
    sid = lax.axis_index("s")
    w = sid * NC + cid
    pltpu.sync_copy(ones_hbm, ones_v)
    pltpu.sync_copy(ei_hbm.at[1, pl.ds(EPW * w, EPW)], didx)
    _row_copy(lambda o, s: zeros_hbm.at[pl.ds(o, s)],
              lambda o, s: acc.at[pl.ds(o, s)], sid)
    plsc.subcore_barrier()

    @pl.loop(0, CPT)
    def _(i):
        pltpu.async_copy(ones_v, acc.at[didx.at[pl.ds(i * CHUNK, CHUNK)]],
                         dsem, add=True)

    @pl.loop(0, CPT)
    def _(i):
        pltpu.make_async_copy(ones_v, acc.at[didx.at[pl.ds(0, CHUNK)]],
                              dsem).wait()

    plsc.subcore_barrier()
    _row_copy(lambda o, s: acc.at[pl.ds(o, s)],
              lambda o, s: out_hbm.at[cid, pl.ds(o, s)], sid)


def _make_agg(C):
    @functools.partial(
        pl.kernel,
        mesh=_MESH,
        out_type=jax.ShapeDtypeStruct((NC, N, C), jnp.bfloat16),
        scratch_types=[
            pltpu.VMEM((EPW,), jnp.int32),
            pltpu.VMEM((EPW,), jnp.int32),
            pltpu.VMEM((CHUNK, C), jnp.bfloat16),
            pltpu.VMEM((CHUNK, C), jnp.bfloat16),
            pltpu.VMEM((CHUNK, C), jnp.bfloat16),
            pltpu.VMEM((CHUNK, C), jnp.bfloat16),
            pltpu.VMEM_SHARED((N, C), jnp.bfloat16),
            pltpu.SemaphoreType.DMA,
            pltpu.SemaphoreType.DMA,
            pltpu.SemaphoreType.DMA,
            pltpu.SemaphoreType.DMA,
            pltpu.SemaphoreType.DMA,
            pltpu.SemaphoreType.DMA,
            pltpu.SemaphoreType.DMA,
            pltpu.SemaphoreType.DMA,
        ],
        compiler_params=pltpu.CompilerParams(use_tc_tiling_on_sc=False),
    )
    def agg_kernel(z_hbm, ei_hbm, zeros_hbm, out_hbm,
                   sidx, didx, rows0, rows1, rows2, rows3, acc,
                   gsem0, gsem1, gsem2, gsem3, ssem0, ssem1, ssem2, ssem3):
        cid = lax.axis_index("c")
        sid = lax.axis_index("s")
        w = sid * NC + cid
        pltpu.sync_copy(ei_hbm.at[0, pl.ds(EPW * w, EPW)], sidx)
        pltpu.sync_copy(ei_hbm.at[1, pl.ds(EPW * w, EPW)], didx)
        _row_copy(lambda o, s: zeros_hbm.at[pl.ds(o, s)],
                  lambda o, s: acc.at[pl.ds(o, s)], sid)
        plsc.subcore_barrier()

        rows = (rows0, rows1, rows2, rows3)
        gsems = (gsem0, gsem1, gsem2, gsem3)
        ssems = (ssem0, ssem1, ssem2, ssem3)

        def gather(i, b):
            pltpu.async_copy(z_hbm.at[sidx.at[pl.ds(i * CHUNK, CHUNK)]],
                             rows[b], gsems[b])

        def wait_g(b):
            pltpu.make_async_copy(z_hbm.at[pl.ds(0, CHUNK)],
                                  rows[b], gsems[b]).wait()

        def scat(i, b):
            pltpu.async_copy(rows[b], acc.at[didx.at[pl.ds(i * CHUNK, CHUNK)]],
                             ssems[b], add=True)

        def wait_s(b):
            pltpu.make_async_copy(rows[b], acc.at[didx.at[pl.ds(0, CHUNK)]],
                                  ssems[b]).wait()

        for b in range(4):
            gather(b, b)

        QUADS = CPT // 4

        @pl.loop(0, QUADS)
        def _(j):
            i0 = 4 * j
            for b in range(4):
                wait_g(b)
                scat(i0 + b, b)
            for b in range(4):
                @pl.when(j < QUADS - 1)
                def _(b=b):
                    wait_s(b)
                    gather(i0 + 4 + b, b)

        wait_s(0)
        gather(CPT - 1, 0)
        wait_g(0)
        scat(CPT - 1, 0)
        wait_s(0)
        wait_s(1)
        wait_s(2)
        wait_s(3)

        plsc.subcore_barrier()
        _row_copy(lambda o, s: acc.at[pl.ds(o, s)],
                  lambda o, s: out_hbm.at[cid, pl.ds(o, s)], sid)

    return agg_kernel


_agg128 = _make_agg(128)
_agg64 = _make_agg(64)


def _dinv_block(degp):
    deg = degp[0, :, 0:1] + degp[1, :, 0:1] + 1.0
    return lax.rsqrt(deg)


def _mm_scale_body(x_ref, w_ref, degp_ref, o_ref):
    o_ref[...] = (jnp.dot(x_ref[...], w_ref[...],
                          preferred_element_type=jnp.float32)
                  * _dinv_block(degp_ref[...])).astype(jnp.bfloat16)


def _tc_mm_scale(x, W, degp):
    return pl.pallas_call(
        _mm_scale_body,
        grid=(N // MB,),
        in_specs=[
            pl.BlockSpec((MB, x.shape[1]), lambda i: (i, 0)),
            pl.BlockSpec(W.shape, lambda i: (0, 0)),
            pl.BlockSpec((NC, MB, 16), lambda i: (0, i, 0)),
        ],
        out_specs=pl.BlockSpec((MB, W.shape[1]), lambda i: (i, 0)),
        out_shape=jax.ShapeDtypeStruct((N, W.shape[1]), jnp.bfloat16),
    )(x, W, degp)


def _layer2_body(z1_ref, p_ref, degp_ref, b1_ref, w2_ref, o_ref):
    dinv = _dinv_block(degp_ref[...])
    agg = (z1_ref[...].astype(jnp.float32) + p_ref[0].astype(jnp.float32)
           + p_ref[1].astype(jnp.float32))
    h = jnp.maximum(agg * dinv + b1_ref[...], 0.0)
    o_ref[...] = (jnp.dot(h, w2_ref[...],
                          preferred_element_type=jnp.float32)
                  * dinv).astype(jnp.bfloat16)


def _tc_layer2(z1, p, degp, b1, W2):
    return pl.pallas_call(
        _layer2_body,
        grid=(N // MB,),
        in_specs=[
            pl.BlockSpec((MB, 128), lambda i: (i, 0)),
            pl.BlockSpec((NC, MB, 128), lambda i: (0, i, 0)),
            pl.BlockSpec((NC, MB, 16), lambda i: (0, i, 0)),
            pl.BlockSpec((1, 128), lambda i: (0, 0)),
            pl.BlockSpec((128, 64), lambda i: (0, 0)),
        ],
        out_specs=pl.BlockSpec((MB, 64), lambda i: (i, 0)),
        out_shape=jax.ShapeDtypeStruct((N, 64), jnp.bfloat16),
    )(z1, p, degp, b1, W2)


def _final_body(z2_ref, q_ref, degp_ref, b2_ref, o_ref):
    dinv = _dinv_block(degp_ref[...])
    out2 = (z2_ref[...].astype(jnp.float32) + q_ref[0].astype(jnp.float32)
            + q_ref[1].astype(jnp.float32)) * dinv + b2_ref[...]
    m = jnp.max(out2, axis=1, keepdims=True)
    e = out2 - m
    lse = jnp.log(jnp.sum(jnp.exp(e), axis=1, keepdims=True))
    o_ref[...] = e - lse


def _tc_final(z2, q, degp, b2):
    return pl.pallas_call(
        _final_body,
        grid=(N // MB,),
        in_specs=[
            pl.BlockSpec((MB, 64), lambda i: (i, 0)),
            pl.BlockSpec((NC, MB, 64), lambda i: (0, i, 0)),
            pl.BlockSpec((NC, MB, 16), lambda i: (0, i, 0)),
            pl.BlockSpec((1, 64), lambda i: (0, 0)),
        ],
        out_specs=pl.BlockSpec((MB, 64), lambda i: (i, 0)),
        out_shape=jax.ShapeDtypeStruct((N, 64), jnp.float32),
    )(z2, q, degp, b2)


def kernel(x, edge_index, W1, b1, W2, b2):
    ones16 = jnp.ones((CHUNK, 16), jnp.float32)
    zeros16 = jnp.zeros((N, 16), jnp.float32)
    zeros128 = jnp.zeros((N, 128), jnp.bfloat16)
    zeros64 = jnp.zeros((N, 64), jnp.bfloat16)

    degp = _deg_kernel(edge_index, ones16, zeros16)
    z1 = _tc_mm_scale(x, W1, degp)
    p = _agg128(z1, edge_index, zeros128)
    z2 = _tc_layer2(z1, p, degp, b1.reshape(1, 128), W2)
    q = _agg64(z2, edge_index, zeros64)
    return _tc_final(z2, q, degp, b2.reshape(1, 64))
